# baseline (speedup 1.0000x reference)
import functools

import jax
import jax.numpy as jnp
import numpy as np
from jax.experimental import pallas as pl
from jax.experimental.pallas import tpu as pltpu


def _toi_tc_kernel(s_ref, e_ref, f_ref, o_ref, fb_ref, *, d, t_len, nh):
    h = pl.program_id(1)

    @pl.when(h == 0)
    def _():
        fb_ref[...] = f_ref[0].astype(jnp.bfloat16)

    s = s_ref[0, pl.ds(h * nh, nh), 0:1]
    e = e_ref[0, pl.ds(h * nh, nh), 0:1]
    col = jax.lax.broadcasted_iota(jnp.int32, (nh, t_len), 1)
    t1 = col - s
    length = e - s
    in_span = (t1 >= 0) & (t1 < length)
    inv_len = 1.0 / length.astype(jnp.float32)
    m_avg = jnp.where(in_span, inv_len, 0.0).astype(jnp.bfloat16)
    m_s = (t1 == 0).astype(jnp.bfloat16)
    m_e = (t1 == length - 1).astype(jnp.bfloat16)
    fb = fb_ref[...]
    dn = (((1,), (1,)), ((), ()))
    o_ref[0, :, 0:d] = jax.lax.dot_general(
        m_s, fb, dn, preferred_element_type=jnp.float32
    )
    o_ref[0, :, d : 2 * d] = jax.lax.dot_general(
        m_avg, fb, dn, preferred_element_type=jnp.float32
    )
    o_ref[0, :, 2 * d : 3 * d] = jax.lax.dot_general(
        m_e, fb, dn, preferred_element_type=jnp.float32
    )


@jax.jit
def kernel(features, tois):
    b, d, t_len = features.shape
    n = tois.shape[1]
    nh = n // 2
    s_col = jnp.broadcast_to(tois[:, :, 0:1], (b, n, 128))
    e_col = jnp.broadcast_to(tois[:, :, 1:2], (b, n, 128))
    out = pl.pallas_call(
        functools.partial(_toi_tc_kernel, d=d, t_len=t_len, nh=nh),
        grid=(b, 2),
        in_specs=[
            pl.BlockSpec((1, n, 128), lambda i, j: (i, 0, 0)),
            pl.BlockSpec((1, n, 128), lambda i, j: (i, 0, 0)),
            pl.BlockSpec((1, d, t_len), lambda i, j: (i, 0, 0)),
        ],
        out_specs=pl.BlockSpec((1, nh, 3 * d), lambda i, j: (2 * i + j, 0, 0)),
        out_shape=jax.ShapeDtypeStruct((2 * b, nh, 3 * d), jnp.float32),
        scratch_shapes=[pltpu.VMEM((d, t_len), jnp.bfloat16)],
    )(s_col, e_col, features)
    offsets = jnp.arange(1, b + 1, dtype=jnp.int32) * np.int32(n)
    return out.reshape(b * n, 3 * d), offsets


# speedup vs baseline: 1.0009x; 1.0009x over previous
import functools

import jax
import jax.numpy as jnp
import numpy as np
from jax.experimental import pallas as pl
from jax.experimental.pallas import tpu as pltpu


def _toi_tc_kernel(s_ref, e_ref, f_ref, o_ref, fb_ref, *, d, t_len, nq):
    h = pl.program_id(1)

    @pl.when(h == 0)
    def _():
        fb_ref[...] = f_ref[0].astype(jnp.bfloat16)

    s = s_ref[0, :, pl.ds(h * nq, nq)]
    e = e_ref[0, :, pl.ds(h * nq, nq)]
    col = jax.lax.broadcasted_iota(jnp.int32, (t_len, nq), 0)
    t1 = col - s
    length = e - s
    in_span = (t1 >= 0) & (t1 < length)
    inv_len = 1.0 / length.astype(jnp.float32)
    m_avg = jnp.where(in_span, inv_len, 0.0).astype(jnp.bfloat16)
    m_s = (t1 == 0).astype(jnp.bfloat16)
    m_e = (t1 == length - 1).astype(jnp.bfloat16)
    m = jnp.concatenate([m_s, m_avg, m_e], axis=1)
    dn = (((0,), (1,)), ((), ()))
    res = jax.lax.dot_general(
        m, fb_ref[...], dn, preferred_element_type=jnp.float32
    )
    o_ref[0, :, 0:d] = res[0:nq]
    o_ref[0, :, d : 2 * d] = res[nq : 2 * nq]
    o_ref[0, :, 2 * d : 3 * d] = res[2 * nq : 3 * nq]


@jax.jit
def kernel(features, tois):
    b, d, t_len = features.shape
    n = tois.shape[1]
    nq = 256
    nsplit = n // nq
    out = pl.pallas_call(
        functools.partial(_toi_tc_kernel, d=d, t_len=t_len, nq=nq),
        grid=(b, nsplit),
        in_specs=[
            pl.BlockSpec((1, 1, n), lambda i, j: (i, 0, 0)),
            pl.BlockSpec((1, 1, n), lambda i, j: (i, 0, 0)),
            pl.BlockSpec((1, d, t_len), lambda i, j: (i, 0, 0)),
        ],
        out_specs=pl.BlockSpec((1, nq, 3 * d), lambda i, j: (nsplit * i + j, 0, 0)),
        out_shape=jax.ShapeDtypeStruct((nsplit * b, nq, 3 * d), jnp.float32),
        scratch_shapes=[pltpu.VMEM((d, t_len), jnp.bfloat16)],
    )(
        tois[:, :, 0].reshape(b, 1, n),
        tois[:, :, 1].reshape(b, 1, n),
        features,
    )
    offsets = jnp.arange(1, b + 1, dtype=jnp.int32) * np.int32(n)
    return out.reshape(b * n, 3 * d), offsets


# speedup vs baseline: 1.0656x; 1.0647x over previous
import functools

import jax
import jax.numpy as jnp
import numpy as np
from jax.experimental import pallas as pl
from jax.experimental.pallas import tpu as pltpu


def _toi_tc_kernel(s_ref, e_ref, f_ref, o_ref, fb_ref, *, d, t_len, nh):
    h = pl.program_id(1)

    @pl.when(h == 0)
    def _():
        fb_ref[...] = f_ref[0].astype(jnp.bfloat16)

    s = s_ref[0, :, pl.ds(h * nh, nh)]
    e = e_ref[0, :, pl.ds(h * nh, nh)]
    col = jax.lax.broadcasted_iota(jnp.int32, (t_len, nh), 0)
    t1 = col - s
    length = e - s
    in_span = (t1 >= 0) & (t1 < length)
    inv_len = 1.0 / length.astype(jnp.float32)
    m_avg = jnp.where(in_span, inv_len, 0.0).astype(jnp.bfloat16)
    m_s = (t1 == 0).astype(jnp.bfloat16)
    m_e = (t1 == length - 1).astype(jnp.bfloat16)
    fb = fb_ref[...]
    dn = (((0,), (1,)), ((), ()))
    o_ref[0, :, 0:d] = jax.lax.dot_general(
        m_s, fb, dn, preferred_element_type=jnp.float32
    )
    o_ref[0, :, d : 2 * d] = jax.lax.dot_general(
        m_avg, fb, dn, preferred_element_type=jnp.float32
    )
    o_ref[0, :, 2 * d : 3 * d] = jax.lax.dot_general(
        m_e, fb, dn, preferred_element_type=jnp.float32
    )


@jax.jit
def kernel(features, tois):
    b, d, t_len = features.shape
    n = tois.shape[1]
    nh = n // 2
    out = pl.pallas_call(
        functools.partial(_toi_tc_kernel, d=d, t_len=t_len, nh=nh),
        grid=(b, 2),
        in_specs=[
            pl.BlockSpec((1, 1, n), lambda i, j: (i, 0, 0)),
            pl.BlockSpec((1, 1, n), lambda i, j: (i, 0, 0)),
            pl.BlockSpec((1, d, t_len), lambda i, j: (i, 0, 0)),
        ],
        out_specs=pl.BlockSpec((1, nh, 3 * d), lambda i, j: (2 * i + j, 0, 0)),
        out_shape=jax.ShapeDtypeStruct((2 * b, nh, 3 * d), jnp.float32),
        scratch_shapes=[pltpu.VMEM((d, t_len), jnp.bfloat16)],
    )(
        tois[:, :, 0].reshape(b, 1, n),
        tois[:, :, 1].reshape(b, 1, n),
        features,
    )
    offsets = jnp.arange(1, b + 1, dtype=jnp.int32) * np.int32(n)
    return out.reshape(b * n, 3 * d), offsets


# speedup vs baseline: 1.0953x; 1.0278x over previous
import functools

import jax
import jax.numpy as jnp
import numpy as np
from jax.experimental import pallas as pl
from jax.experimental.pallas import tpu as pltpu


def _toi_tc_kernel(s_ref, e_ref, f_ref, o_ref, *, d: int, t_len: int, nh: int):
    h = pl.program_id(1)
    f = f_ref[0]
    s = s_ref[0, :, pl.ds(h * nh, nh)]
    e = e_ref[0, :, pl.ds(h * nh, nh)]
    col = jax.lax.broadcasted_iota(jnp.int32, (t_len, nh), 0)
    in_span = (col >= s) & (col < e)
    inv_len = 1.0 / (e - s).astype(jnp.float32)
    fb = f.astype(jnp.bfloat16)
    m_avg = jnp.where(in_span, inv_len, 0.0).astype(jnp.bfloat16)
    m_s = (col == s).astype(jnp.bfloat16)
    m_e = (col == e - 1).astype(jnp.bfloat16)
    dn = (((0,), (1,)), ((), ()))
    o_ref[0, :, 0:d] = jax.lax.dot_general(
        m_s, fb, dn, preferred_element_type=jnp.float32
    )
    o_ref[0, :, d : 2 * d] = jax.lax.dot_general(
        m_avg, fb, dn, preferred_element_type=jnp.float32
    )
    o_ref[0, :, 2 * d : 3 * d] = jax.lax.dot_general(
        m_e, fb, dn, preferred_element_type=jnp.float32
    )


@jax.jit
def kernel(features, tois):
    b, d, t_len = features.shape
    n = tois.shape[1]
    nh = n // 2
    out = pl.pallas_call(
        functools.partial(_toi_tc_kernel, d=d, t_len=t_len, nh=nh),
        grid=(b, 2),
        in_specs=[
            pl.BlockSpec((1, 1, n), lambda i, j: (i, 0, 0)),
            pl.BlockSpec((1, 1, n), lambda i, j: (i, 0, 0)),
            pl.BlockSpec((1, d, t_len), lambda i, j: (i, 0, 0)),
        ],
        out_specs=pl.BlockSpec((1, nh, 3 * d), lambda i, j: (2 * i + j, 0, 0)),
        out_shape=jax.ShapeDtypeStruct((2 * b, nh, 3 * d), jnp.float32),
    )(
        tois[:, :, 0].reshape(b, 1, n),
        tois[:, :, 1].reshape(b, 1, n),
        features,
    )
    offsets = jnp.arange(1, b + 1, dtype=jnp.int32) * np.int32(n)
    return out.reshape(b * n, 3 * d), offsets


# speedup vs baseline: 1.1104x; 1.0138x over previous
import functools

import jax
import jax.numpy as jnp
import numpy as np
from jax.experimental import pallas as pl


def _toi_tc_kernel(s_ref, e_ref, f_ref, o_ref, *, d: int, t_len: int, n: int):
    f = f_ref[0]
    s = s_ref[0]
    e = e_ref[0]
    col = jax.lax.broadcasted_iota(jnp.int32, (t_len, n), 0)
    in_span = (col >= s) & (col < e)
    inv_len = 1.0 / (e - s).astype(jnp.float32)
    fb = f.astype(jnp.bfloat16)
    m_avg = jnp.where(in_span, inv_len, 0.0).astype(jnp.bfloat16)
    m_s = (col == s).astype(jnp.bfloat16)
    m_e = (col == e - 1).astype(jnp.bfloat16)
    dn = (((0,), (1,)), ((), ()))
    o_ref[0, :, 0:d] = jax.lax.dot_general(
        m_s, fb, dn, preferred_element_type=jnp.float32
    )
    o_ref[0, :, d : 2 * d] = jax.lax.dot_general(
        m_avg, fb, dn, preferred_element_type=jnp.float32
    )
    o_ref[0, :, 2 * d : 3 * d] = jax.lax.dot_general(
        m_e, fb, dn, preferred_element_type=jnp.float32
    )


@jax.jit
def kernel(features, tois):
    b, d, t_len = features.shape
    n = tois.shape[1]
    out = pl.pallas_call(
        functools.partial(_toi_tc_kernel, d=d, t_len=t_len, n=n),
        grid=(b,),
        in_specs=[
            pl.BlockSpec((1, 1, n), lambda i: (i, 0, 0)),
            pl.BlockSpec((1, 1, n), lambda i: (i, 0, 0)),
            pl.BlockSpec((1, d, t_len), lambda i: (i, 0, 0)),
        ],
        out_specs=pl.BlockSpec((1, n, 3 * d), lambda i: (i, 0, 0)),
        out_shape=jax.ShapeDtypeStruct((b, n, 3 * d), jnp.float32),
    )(
        tois[:, :, 0].reshape(b, 1, n),
        tois[:, :, 1].reshape(b, 1, n),
        features,
    )
    offsets = jnp.arange(1, b + 1, dtype=jnp.int32) * np.int32(n)
    return out.reshape(b * n, 3 * d), offsets
